# baseline (speedup 1.0000x reference)
import functools

import jax
import jax.numpy as jnp
from jax import lax
from jax.experimental import pallas as pl
from jax.experimental.pallas import tpu as pltpu
from jax.experimental.pallas import tpu_sc as plsc

NW = 32

B = 1024
L = 200
HID = 128
GIP = 2
SCB = 256
FCB_HEADS = 4
FCB = 256
GH = 4
GD = 32
ITEM_NUM = 100000
TAX_NUM = 1000
FIRST = 10
PER_TAX = 20
N_ITEM = B * L
N_TAX = B * PER_TAX
E_I2T = N_ITEM
E_T2T = 2 * N_TAX
TW = 144


def _gelu(x):
    return x * 0.5 * (1.0 + lax.erf(x * (2.0 ** -0.5)))


def _lnT(xT, s, b, eps=1e-8):
    m = jnp.mean(xT, axis=0, keepdims=True)
    v = jnp.mean((xT - m) ** 2, axis=0, keepdims=True)
    return (xT - m) / jnp.sqrt(v + eps) * s[:, None] + b[:, None]


TOWER_BB = 8


def _tower_body(v_ref, *refs):
    out_ref = refs[-1]
    wv = refs[-2][...]
    for bb in range(TOWER_BB):
        vT = v_ref[bb].T
        for blk in range(GIP):
            (ln1s, ln1b, w1, w2, ln2s, ln2b, W1bd, W2bd, w3) = refs[blk * 9:(blk + 1) * 9]
            nVT = _lnT(vT, ln1s[...], ln1b[...])
            t1 = jnp.dot(nVT, w1[...], preferred_element_type=jnp.float32)
            scbT = jnp.dot(_gelu(t1), w2[...], preferred_element_type=jnp.float32)
            vsT = nVT + scbT
            nVsT = _lnT(vsT, ln2s[...], ln2b[...])
            c1 = jnp.dot(W1bd[...].T, nVsT, preferred_element_type=jnp.float32)
            c2 = jnp.dot(W2bd[...].T, _gelu(c1), preferred_element_type=jnp.float32)
            vT = nVsT + jnp.dot(w3[...].T, c2, preferred_element_type=jnp.float32)
        s = jnp.sum(vT * wv, axis=0, keepdims=True)
        s = s - jnp.max(s, axis=1, keepdims=True)
        e = jnp.exp(s)
        alpha = e / jnp.sum(e, axis=1, keepdims=True)
        out_ref[bb, :] = jnp.sum(vT * alpha, axis=1)


def _seq_tower(V, blocks, wv):
    wrefs = []
    for blk in blocks:
        W1bd = jnp.zeros((HID, FCB_HEADS * FCB), jnp.float32)
        W2bd = jnp.zeros((FCB_HEADS * FCB, HID), jnp.float32)
        hd = HID // FCB_HEADS
        for h in range(FCB_HEADS):
            W1bd = W1bd.at[h * hd:(h + 1) * hd, h * FCB:(h + 1) * FCB].set(blk['fcb_w1'])
            W2bd = W2bd.at[h * FCB:(h + 1) * FCB, h * hd:(h + 1) * hd].set(blk['fcb_w2'])
        wrefs += [blk['scb_ln_s'], blk['scb_ln_b'], blk['scb_w1'], blk['scb_w2'],
                  blk['fcb_ln_s'], blk['fcb_ln_b'], W1bd, W2bd, blk['fcb_w3']]
    wrefs.append(wv)
    n_w = len(wrefs)
    in_specs = [pl.BlockSpec((TOWER_BB, L, HID), lambda i: (i, 0, 0))]
    in_specs += [pl.BlockSpec(w.shape, lambda i, nd=w.ndim: (0,) * nd) for w in wrefs]
    return pl.pallas_call(
        _tower_body,
        grid=(B // TOWER_BB,),
        in_specs=in_specs,
        out_specs=pl.BlockSpec((TOWER_BB, HID), lambda i: (i, 0)),
        out_shape=jax.ShapeDtypeStruct((B, HID), jnp.float32),
    )(V, *wrefs)


def _mk_src_tab_body(x_ref, w_ref, al_ref, out_ref):
    x = x_ref[...]
    h = jnp.dot(x, w_ref[...], preferred_element_type=jnp.float32)
    hr = h.reshape(h.shape[0], GH, GD)
    el4 = jnp.sum(hr * al_ref[...][None], axis=-1)
    z = jnp.zeros((h.shape[0], 62), jnp.float32)
    half0 = jnp.concatenate([h[:, :64], el4[:, 0:2], z], axis=1)
    half1 = jnp.concatenate([h[:, 64:], el4[:, 2:4], z], axis=1)
    out_ref[...] = jnp.stack([half0, half1], axis=1)


def _mk_er_tab_body(x_ref, w_ref, ar_ref, out_ref, out2_ref):
    x = x_ref[...]
    h = jnp.dot(x, w_ref[...], preferred_element_type=jnp.float32)
    hr = h.reshape(h.shape[0], GH, GD)
    er4 = jnp.sum(hr * ar_ref[...][None], axis=-1)
    z = jnp.zeros((h.shape[0], 14), jnp.float32)
    out_ref[...] = jnp.concatenate([er4[:, 0:2], z], axis=1)
    out2_ref[...] = jnp.concatenate([er4[:, 2:4], z], axis=1)


def _mk_src_tab(x, W, al, chunk):
    n = x.shape[0]
    return pl.pallas_call(
        _mk_src_tab_body,
        grid=(n // chunk,),
        in_specs=[pl.BlockSpec((chunk, HID), lambda i: (i, 0)),
                  pl.BlockSpec(W.shape, lambda i: (0, 0)),
                  pl.BlockSpec(al.shape, lambda i: (0, 0))],
        out_specs=pl.BlockSpec((chunk, 2, HID), lambda i: (i, 0, 0)),
        out_shape=jax.ShapeDtypeStruct((n, 2, HID), jnp.float32),
    )(x, W, al)


def _mk_er_tab(x, W, ar, chunk):
    n = x.shape[0]
    return pl.pallas_call(
        _mk_er_tab_body,
        grid=(n // chunk,),
        in_specs=[pl.BlockSpec((chunk, HID), lambda i: (i, 0)),
                  pl.BlockSpec(W.shape, lambda i: (0, 0)),
                  pl.BlockSpec(ar.shape, lambda i: (0, 0))],
        out_specs=[pl.BlockSpec((chunk, 16), lambda i: (i, 0)),
                   pl.BlockSpec((chunk, 16), lambda i: (i, 0))],
        out_shape=[jax.ShapeDtypeStruct((n, 16), jnp.float32),
                   jax.ShapeDtypeStruct((n, 16), jnp.float32)],
    )(x, W, ar)


CK = 128
AW = 80
DUMP = 128


@functools.lru_cache(maxsize=None)
def _mk_gat_edge(N_src, N_dst, E, half):
    NT = 16
    e_per_tile = E // NT
    n_chunks = e_per_tile // CK
    NH = N_dst // 2
    NA = NH + DUMP
    zrows_pt = NA // NT
    worows_pt = NH // NT
    lo = half * NH
    assert E % (NT * CK) == 0 and N_dst % (2 * NT) == 0 and NA % NT == 0
    mesh = plsc.VectorSubcoreMesh(core_axis_name="c", subcore_axis_name="s")
    i32 = jnp.int32

    @functools.partial(
        pl.kernel, mesh=mesh,
        out_type=jax.ShapeDtypeStruct((2 * NH, AW), jnp.float32),
        scratch_types=[
            pltpu.VMEM((CK,), i32),
            pltpu.VMEM((CK,), i32),
            pltpu.VMEM((CK,), i32),
            pltpu.VMEM((CK,), i32),
            pltpu.VMEM((CK, HID), jnp.float32),
            pltpu.VMEM((CK, 16), jnp.float32),
            pltpu.VMEM((CK, AW), jnp.float32),
            pltpu.VMEM_SHARED((NA, AW), jnp.float32),
            pltpu.VMEM_SHARED((N_dst, 16), jnp.float32),
            pltpu.SemaphoreType.DMA,
            pltpu.SemaphoreType.DMA,
        ],
    )
    def k(stab_hbm, ertA_hbm, ertB_hbm, src_hbm, dst_hbm, out_hbm,
          sidx, didx, sidxc, didxl, rows_v, er_v, vals_v, acc_sh, er_sh,
          sem, sem2):
        c = lax.axis_index("c")
        tid = lax.axis_index("s")
        iota16 = lax.iota(i32, 16)
        zero16 = jnp.zeros((16,), jnp.float32)

        ern = N_dst // NT
        erb = tid * ern
        for q in range(ern // CK):
            @pl.when(c == 0)
            def _():
                pltpu.sync_copy(ertA_hbm.at[pl.ds(erb + q * CK, CK)], er_v)

            @pl.when(c == 1)
            def _():
                pltpu.sync_copy(ertB_hbm.at[pl.ds(erb + q * CK, CK)], er_v)

            pltpu.sync_copy(er_v, er_sh.at[pl.ds(erb + q * CK, CK)])

        def zv(r, carry):
            for g in range(AW // 16):
                vals_v[r, pl.ds(g * 16, 16)] = zero16
            return carry
        lax.fori_loop(0, CK, zv, 0)
        zbase = tid * zrows_pt
        done = 0
        while done < zrows_pt:
            step = min(CK, zrows_pt - done)
            pltpu.sync_copy(vals_v.at[pl.ds(0, step)],
                            acc_sh.at[pl.ds(zbase + done, step)])
            done += step
        plsc.subcore_barrier()

        ebase = tid * e_per_tile

        def chunk_body(q, carry):
            off = ebase + q * CK
            pltpu.sync_copy(src_hbm.at[pl.ds(off, CK)], sidx)
            pltpu.sync_copy(dst_hbm.at[pl.ds(off, CK)], didx)

            def mkidx(g, carry2):
                sidxc[pl.ds(g * 16, 16)] = sidx[pl.ds(g * 16, 16)] * 2 + c
                return carry2

            lax.fori_loop(0, CK // 16, mkidx, 0)
            pltpu.async_copy(stab_hbm.at[sidxc], rows_v, sem).wait()
            pltpu.async_copy(er_sh.at[didx], er_v, sem2).wait()

            def route(g, carry2):
                d = didx[pl.ds(g * 16, 16)]
                dl = d - lo
                m = (dl >= 0) & (dl < NH)
                didxl[pl.ds(g * 16, 16)] = jnp.where(
                    m, dl, NH + (iota16 & (DUMP - 1)))
                return carry2

            lax.fori_loop(0, CK // 16, route, 0)

            def per_edge(e, carry2):
                erow = er_v[e, pl.ds(0, 16)]
                t = rows_v[e, pl.ds(64, 16)]
                z = t + erow
                ev = jnp.exp(jnp.maximum(z, 0.2 * z))
                va = zero16 + ev[0]
                vb = zero16 + ev[1]
                vals_v[e, pl.ds(0, 16)] = rows_v[e, pl.ds(0, 16)] * va
                vals_v[e, pl.ds(16, 16)] = rows_v[e, pl.ds(16, 16)] * va
                vals_v[e, pl.ds(32, 16)] = rows_v[e, pl.ds(32, 16)] * vb
                vals_v[e, pl.ds(48, 16)] = rows_v[e, pl.ds(48, 16)] * vb
                vals_v[e, pl.ds(64, 16)] = jnp.where(
                    iota16 == 0, va, jnp.where(iota16 == 1, vb, 0.0))
                return carry2

            lax.fori_loop(0, CK, per_edge, 0)
            pltpu.sync_copy(vals_v, acc_sh.at[didxl], add=True)
            return carry

        lax.fori_loop(0, n_chunks, chunk_body, 0)
        plsc.subcore_barrier()

        wbase = tid * worows_pt
        for q in range(worows_pt // CK):
            pltpu.sync_copy(acc_sh.at[pl.ds(wbase + q * CK, CK)], vals_v)
            pltpu.sync_copy(vals_v,
                            out_hbm.at[pl.ds(c * NH + wbase + q * CK, CK)])

    return k


def _gat_edge_sc(src_tab3, er_tabs, src, dst, n_dst):
    rows = src_tab3[src]
    el = jnp.concatenate([rows[:, 0, 64:66], rows[:, 1, 64:66]], axis=-1)
    er = jnp.concatenate([er_tabs[0][dst][:, :2], er_tabs[1][dst][:, :2]],
                         axis=-1)
    z = el + er
    ex = jnp.exp(jnp.maximum(z, 0.2 * z))
    accs = []
    for cc in range(2):
        exc = ex[:, 2 * cc:2 * cc + 2]
        vals = jnp.concatenate([
            rows[:, cc, :64] * jnp.repeat(exc, 64 // 2, axis=1),
            exc, jnp.zeros((src.shape[0], AW - 66), jnp.float32)], axis=1)
        accs.append(jax.ops.segment_sum(vals, dst, num_segments=n_dst))
    return jnp.stack(accs, axis=0)


def _norm_body(acc1_ref, acc2_ref, b1_ref, b2_ref, exp_ref, out_ref):
    expm = exp_ref[...]
    def one(acc_ref, b_ref):
        halves = []
        for c in range(2):
            acc = acc_ref[c]
            den = acc[:, 64:66]
            rec = 1.0 / (den + 1e-9)
            recx = jnp.dot(rec, expm, preferred_element_type=jnp.float32)
            halves.append(acc[:, :64] * recx)
        return jnp.concatenate(halves, axis=1) + b_ref[...][None]
    out_ref[...] = one(acc1_ref, b1_ref) + one(acc2_ref, b2_ref)


def _norm_combine(acc1, acc2, b1, b2, chunk=2048):
    n = acc1.shape[1]
    expm = jnp.zeros((2, 64), jnp.float32)
    for h in range(2):
        expm = expm.at[h, h * GD:(h + 1) * GD].set(1.0)
    return pl.pallas_call(
        _norm_body,
        grid=(n // chunk,),
        in_specs=[pl.BlockSpec((2, chunk, AW), lambda i: (0, i, 0)),
                  pl.BlockSpec((2, chunk, AW), lambda i: (0, i, 0)),
                  pl.BlockSpec((HID,), lambda i: (0,)),
                  pl.BlockSpec((HID,), lambda i: (0,)),
                  pl.BlockSpec((2, 64), lambda i: (0, 0))],
        out_specs=pl.BlockSpec((chunk, HID), lambda i: (i, 0)),
        out_shape=jax.ShapeDtypeStruct((n, HID), jnp.float32),
    )(acc1, acc2, b1, b2, expm)


def _final_body(local_ref, gint_ref, pos_ref, neg_ref, out_ref):
    local = local_ref[...]
    g = gint_ref[...]
    mul = jnp.sum(local * g[:, None, :], axis=-1)
    masked = jnp.where(mul != 0, mul, -jnp.inf)
    m = jnp.max(masked, axis=-1, keepdims=True)
    e = jnp.exp(masked - m)
    w = e / jnp.sum(e, axis=-1, keepdims=True)
    intention = g + jnp.sum(w[:, :, None] * local, axis=1)
    out_ref[0, :, :] = jnp.stack([
        jnp.sum(intention * pos_ref[...], axis=-1),
        jnp.sum(intention * neg_ref[...], axis=-1)], axis=0)


def _final(local, g_int, pos_e, neg_e, chunk=128):
    out = pl.pallas_call(
        _final_body,
        grid=(B // chunk,),
        in_specs=[pl.BlockSpec((chunk, FIRST, HID), lambda i: (i, 0, 0)),
                  pl.BlockSpec((chunk, HID), lambda i: (i, 0)),
                  pl.BlockSpec((chunk, HID), lambda i: (i, 0)),
                  pl.BlockSpec((chunk, HID), lambda i: (i, 0))],
        out_specs=pl.BlockSpec((1, 2, chunk), lambda i: (i, 0, 0)),
        out_shape=jax.ShapeDtypeStruct((B // chunk, 2, chunk), jnp.float32),
    )(local, g_int, pos_e, neg_e)
    out = jnp.swapaxes(out, 0, 1).reshape(2, B)
    return out[0], out[1]


@functools.lru_cache(maxsize=None)
def _mk_sc_gather(V, D, N, chunk):
    assert N % (8 * NW) == 0 and (N // NW) % chunk == 0 and chunk % 8 == 0
    b_per_w = N // NW
    n_iter = b_per_w // chunk
    mesh = plsc.VectorSubcoreMesh(core_axis_name="c", subcore_axis_name="s")

    @functools.partial(
        pl.kernel, mesh=mesh,
        out_type=jax.ShapeDtypeStruct((N, D), jnp.float32),
        scratch_types=[
            pltpu.VMEM((chunk,), jnp.int32),
            pltpu.VMEM((chunk, D), jnp.float32),
            pltpu.SemaphoreType.DMA,
        ],
    )
    def k(table_hbm, idx_hbm, out_hbm, idx_v, rows_v, sem):
        wid = lax.axis_index("s") * 2 + lax.axis_index("c")
        base = wid * b_per_w

        def body(i, carry):
            off = base + i * chunk
            pltpu.sync_copy(idx_hbm.at[pl.ds(off, chunk)], idx_v)
            pltpu.async_copy(table_hbm.at[idx_v], rows_v, sem).wait()
            pltpu.sync_copy(rows_v, out_hbm.at[pl.ds(off, chunk)])
            return carry

        lax.fori_loop(0, n_iter, body, 0)

    return k


def _gather_rows(table, idx, chunk=640):
    return _mk_sc_gather(table.shape[0], table.shape[1], idx.shape[0], chunk)(table, idx)


def kernel(params, seq, pos, neg, root, item_ids, tax_ids,
           i2t_src, i2t_dst, t2t_src, t2t_dst, batch_num_tax):
    item_embed = params['item_embed']
    tax_embed = params['tax_embed']

    V = _gather_rows(item_embed, seq.reshape(-1)).reshape(B, L, HID)
    g_int = _seq_tower(V, params['blocks'], params['wv'])

    item_h = _gather_rows(item_embed, item_ids)
    tax_h = _gather_rows(tax_embed, tax_ids)
    for lyr in params['gnn']:
        ali = lyr['ali']
        art_i = lyr['ari']
        src_tab_i = _mk_src_tab(item_h, lyr['Wi'], ali, 2048)
        er_tab_i = _mk_er_tab(tax_h, lyr['Wi'], art_i, 2048)
        src_tab_t = _mk_src_tab(tax_h, lyr['Wt'], lyr['alt'], 2048)
        er_tab_t = _mk_er_tab(tax_h, lyr['Wt'], lyr['art'], 2048)
        acc_i = _gat_edge_sc(src_tab_i, er_tab_i, i2t_src, i2t_dst, N_TAX)
        acc_t = _gat_edge_sc(src_tab_t, er_tab_t, t2t_src, t2t_dst, N_TAX)
        tax_h = _norm_combine(acc_i, acc_t, lyr['bi'], lyr['bt'])

    tmp = jnp.roll(jnp.cumsum(batch_num_tax), 1).at[0].set(0)
    root_idx = (root + tmp[:, None]).reshape(-1)
    local = _gather_rows(tax_h, root_idx, chunk=320).reshape(B, FIRST, HID)
    valid = (root != -1)
    local = jnp.where(valid[:, :, None], local, 0.0)
    pn = _gather_rows(item_embed, jnp.concatenate([pos, neg]), chunk=64)
    pos_e, neg_e = pn[:B], pn[B:]
    return _final(local, g_int, pos_e, neg_e)


# speedup vs baseline: 1.6595x; 1.6595x over previous
import functools

import jax
import jax.numpy as jnp
from jax import lax
from jax.experimental import pallas as pl
from jax.experimental.pallas import tpu as pltpu
from jax.experimental.pallas import tpu_sc as plsc

NW = 32

B = 1024
L = 200
HID = 128
GIP = 2
SCB = 256
FCB_HEADS = 4
FCB = 256
GH = 4
GD = 32
ITEM_NUM = 100000
TAX_NUM = 1000
FIRST = 10
PER_TAX = 20
N_ITEM = B * L
N_TAX = B * PER_TAX
E_I2T = N_ITEM
E_T2T = 2 * N_TAX
TW = 144


def _gelu(x):
    return x * 0.5 * (1.0 + lax.erf(x * (2.0 ** -0.5)))


def _lnT(xT, s, b, eps=1e-8):
    m = jnp.mean(xT, axis=0, keepdims=True)
    v = jnp.mean((xT - m) ** 2, axis=0, keepdims=True)
    return (xT - m) / jnp.sqrt(v + eps) * s[:, None] + b[:, None]


TOWER_BB = 8


def _tower_body(v_ref, *refs):
    out_ref = refs[-1]
    wv = refs[-2][...]
    for bb in range(TOWER_BB):
        vT = v_ref[bb].T
        for blk in range(GIP):
            (ln1s, ln1b, w1, w2, ln2s, ln2b, W1bd, W2bd, w3) = refs[blk * 9:(blk + 1) * 9]
            nVT = _lnT(vT, ln1s[...], ln1b[...])
            t1 = jnp.dot(nVT, w1[...], preferred_element_type=jnp.float32)
            scbT = jnp.dot(_gelu(t1), w2[...], preferred_element_type=jnp.float32)
            vsT = nVT + scbT
            nVsT = _lnT(vsT, ln2s[...], ln2b[...])
            c1 = jnp.dot(W1bd[...].T, nVsT, preferred_element_type=jnp.float32)
            c2 = jnp.dot(W2bd[...].T, _gelu(c1), preferred_element_type=jnp.float32)
            vT = nVsT + jnp.dot(w3[...].T, c2, preferred_element_type=jnp.float32)
        s = jnp.sum(vT * wv, axis=0, keepdims=True)
        s = s - jnp.max(s, axis=1, keepdims=True)
        e = jnp.exp(s)
        alpha = e / jnp.sum(e, axis=1, keepdims=True)
        out_ref[bb, :] = jnp.sum(vT * alpha, axis=1)


def _seq_tower(V, blocks, wv):
    wrefs = []
    for blk in blocks:
        W1bd = jnp.zeros((HID, FCB_HEADS * FCB), jnp.float32)
        W2bd = jnp.zeros((FCB_HEADS * FCB, HID), jnp.float32)
        hd = HID // FCB_HEADS
        for h in range(FCB_HEADS):
            W1bd = W1bd.at[h * hd:(h + 1) * hd, h * FCB:(h + 1) * FCB].set(blk['fcb_w1'])
            W2bd = W2bd.at[h * FCB:(h + 1) * FCB, h * hd:(h + 1) * hd].set(blk['fcb_w2'])
        wrefs += [blk['scb_ln_s'], blk['scb_ln_b'], blk['scb_w1'], blk['scb_w2'],
                  blk['fcb_ln_s'], blk['fcb_ln_b'], W1bd, W2bd, blk['fcb_w3']]
    wrefs.append(wv)
    n_w = len(wrefs)
    in_specs = [pl.BlockSpec((TOWER_BB, L, HID), lambda i: (i, 0, 0))]
    in_specs += [pl.BlockSpec(w.shape, lambda i, nd=w.ndim: (0,) * nd) for w in wrefs]
    return pl.pallas_call(
        _tower_body,
        grid=(B // TOWER_BB,),
        in_specs=in_specs,
        out_specs=pl.BlockSpec((TOWER_BB, HID), lambda i: (i, 0)),
        out_shape=jax.ShapeDtypeStruct((B, HID), jnp.float32),
    )(V, *wrefs)


def _mk_src_tab_body(x_ref, w_ref, al_ref, out_ref):
    x = x_ref[...]
    h = jnp.dot(x, w_ref[...], preferred_element_type=jnp.float32)
    hr = h.reshape(h.shape[0], GH, GD)
    el4 = jnp.sum(hr * al_ref[...][None], axis=-1)
    ones = jnp.ones((h.shape[0], GH), jnp.float32)
    pad = jnp.zeros((h.shape[0], TW - HID - 2 * GH), jnp.float32)
    out_ref[...] = jnp.concatenate([h, ones, el4, pad], axis=1)


def _mk_er_tab_body(x_ref, w_ref, ar_ref, out_ref):
    x = x_ref[...]
    h = jnp.dot(x, w_ref[...], preferred_element_type=jnp.float32)
    hr = h.reshape(h.shape[0], GH, GD)
    er4 = jnp.sum(hr * ar_ref[...][None], axis=-1)
    pad = jnp.zeros((h.shape[0], 12), jnp.float32)
    out_ref[...] = jnp.concatenate([er4, pad], axis=1)


def _mk_src_tab(x, W, al, chunk):
    n = x.shape[0]
    return pl.pallas_call(
        _mk_src_tab_body,
        grid=(n // chunk,),
        in_specs=[pl.BlockSpec((chunk, HID), lambda i: (i, 0)),
                  pl.BlockSpec(W.shape, lambda i: (0, 0)),
                  pl.BlockSpec(al.shape, lambda i: (0, 0))],
        out_specs=pl.BlockSpec((chunk, TW), lambda i: (i, 0)),
        out_shape=jax.ShapeDtypeStruct((n, TW), jnp.float32),
    )(x, W, al)


def _mk_er_tab(x, W, ar, chunk):
    n = x.shape[0]
    return pl.pallas_call(
        _mk_er_tab_body,
        grid=(n // chunk,),
        in_specs=[pl.BlockSpec((chunk, HID), lambda i: (i, 0)),
                  pl.BlockSpec(W.shape, lambda i: (0, 0)),
                  pl.BlockSpec(ar.shape, lambda i: (0, 0))],
        out_specs=pl.BlockSpec((chunk, 16), lambda i: (i, 0)),
        out_shape=jax.ShapeDtypeStruct((n, 16), jnp.float32),
    )(x, W, ar)


def _gat_edge(src_tab, er_tab, src, dst, n_dst):
    rows = src_tab[src]
    el = rows[:, HID + GH:HID + 2 * GH]
    er = er_tab[dst][:, :GH]
    z = el + er
    e = jnp.maximum(z, 0.2 * z)
    ex = jnp.exp(e)
    mult_main = jnp.repeat(ex, GD, axis=1)
    mult_tail = jnp.tile(ex, (1, (TW - HID) // GH))
    vals = rows * jnp.concatenate([mult_main, mult_tail], axis=1)
    return jax.ops.segment_sum(vals, dst, num_segments=n_dst)


def _norm_body(acc1_ref, acc2_ref, b1_ref, b2_ref, exp_ref, out_ref):
    expm = exp_ref[...]
    def one(acc_ref, b_ref):
        acc = acc_ref[...]
        den = acc[:, HID:HID + GH]
        rec = 1.0 / (den + 1e-9)
        recx = jnp.dot(rec, expm, preferred_element_type=jnp.float32)
        return acc[:, :HID] * recx + b_ref[...][None]
    out_ref[...] = one(acc1_ref, b1_ref) + one(acc2_ref, b2_ref)


def _norm_combine(acc1, acc2, b1, b2, chunk=2048):
    n = acc1.shape[0]
    expm = jnp.zeros((GH, HID), jnp.float32)
    for h in range(GH):
        expm = expm.at[h, h * GD:(h + 1) * GD].set(1.0)
    return pl.pallas_call(
        _norm_body,
        grid=(n // chunk,),
        in_specs=[pl.BlockSpec((chunk, TW), lambda i: (i, 0)),
                  pl.BlockSpec((chunk, TW), lambda i: (i, 0)),
                  pl.BlockSpec((HID,), lambda i: (0,)),
                  pl.BlockSpec((HID,), lambda i: (0,)),
                  pl.BlockSpec((GH, HID), lambda i: (0, 0))],
        out_specs=pl.BlockSpec((chunk, HID), lambda i: (i, 0)),
        out_shape=jax.ShapeDtypeStruct((n, HID), jnp.float32),
    )(acc1, acc2, b1, b2, expm)


def _final_body(local_ref, gint_ref, pos_ref, neg_ref, out_ref):
    local = local_ref[...]
    g = gint_ref[...]
    mul = jnp.sum(local * g[:, None, :], axis=-1)
    masked = jnp.where(mul != 0, mul, -jnp.inf)
    m = jnp.max(masked, axis=-1, keepdims=True)
    e = jnp.exp(masked - m)
    w = e / jnp.sum(e, axis=-1, keepdims=True)
    intention = g + jnp.sum(w[:, :, None] * local, axis=1)
    out_ref[0, :, :] = jnp.stack([
        jnp.sum(intention * pos_ref[...], axis=-1),
        jnp.sum(intention * neg_ref[...], axis=-1)], axis=0)


def _final(local, g_int, pos_e, neg_e, chunk=128):
    out = pl.pallas_call(
        _final_body,
        grid=(B // chunk,),
        in_specs=[pl.BlockSpec((chunk, FIRST, HID), lambda i: (i, 0, 0)),
                  pl.BlockSpec((chunk, HID), lambda i: (i, 0)),
                  pl.BlockSpec((chunk, HID), lambda i: (i, 0)),
                  pl.BlockSpec((chunk, HID), lambda i: (i, 0))],
        out_specs=pl.BlockSpec((1, 2, chunk), lambda i: (i, 0, 0)),
        out_shape=jax.ShapeDtypeStruct((B // chunk, 2, chunk), jnp.float32),
    )(local, g_int, pos_e, neg_e)
    out = jnp.swapaxes(out, 0, 1).reshape(2, B)
    return out[0], out[1]


@functools.lru_cache(maxsize=None)
def _mk_sc_gather(V, D, N, chunk):
    assert N % (8 * NW) == 0 and (N // NW) % chunk == 0 and chunk % 8 == 0
    b_per_w = N // NW
    n_iter = b_per_w // chunk
    mesh = plsc.VectorSubcoreMesh(core_axis_name="c", subcore_axis_name="s")

    @functools.partial(
        pl.kernel, mesh=mesh,
        out_type=jax.ShapeDtypeStruct((N, D), jnp.float32),
        scratch_types=[
            pltpu.VMEM((chunk,), jnp.int32),
            pltpu.VMEM((chunk, D), jnp.float32),
            pltpu.SemaphoreType.DMA,
        ],
    )
    def k(table_hbm, idx_hbm, out_hbm, idx_v, rows_v, sem):
        wid = lax.axis_index("s") * 2 + lax.axis_index("c")
        base = wid * b_per_w

        def body(i, carry):
            off = base + i * chunk
            pltpu.sync_copy(idx_hbm.at[pl.ds(off, chunk)], idx_v)
            pltpu.async_copy(table_hbm.at[idx_v], rows_v, sem).wait()
            pltpu.sync_copy(rows_v, out_hbm.at[pl.ds(off, chunk)])
            return carry

        lax.fori_loop(0, n_iter, body, 0)

    return k


def _gather_rows(table, idx, chunk=640):
    return _mk_sc_gather(table.shape[0], table.shape[1], idx.shape[0], chunk)(table, idx)


def kernel(params, seq, pos, neg, root, item_ids, tax_ids,
           i2t_src, i2t_dst, t2t_src, t2t_dst, batch_num_tax):
    item_embed = params['item_embed']
    tax_embed = params['tax_embed']

    V = _gather_rows(item_embed, seq.reshape(-1)).reshape(B, L, HID)
    g_int = _seq_tower(V, params['blocks'], params['wv'])

    item_h = _gather_rows(item_embed, item_ids)
    tax_h = _gather_rows(tax_embed, tax_ids)
    for lyr in params['gnn']:
        ali = lyr['ali']
        art_i = lyr['ari']
        src_tab_i = _mk_src_tab(item_h, lyr['Wi'], ali, 2048)
        er_tab_i = _mk_er_tab(tax_h, lyr['Wi'], art_i, 2048)
        src_tab_t = _mk_src_tab(tax_h, lyr['Wt'], lyr['alt'], 2048)
        er_tab_t = _mk_er_tab(tax_h, lyr['Wt'], lyr['art'], 2048)
        acc_i = _gat_edge(src_tab_i, er_tab_i, i2t_src, i2t_dst, N_TAX)
        acc_t = _gat_edge(src_tab_t, er_tab_t, t2t_src, t2t_dst, N_TAX)
        tax_h = _norm_combine(acc_i, acc_t, lyr['bi'], lyr['bt'])

    tmp = jnp.roll(jnp.cumsum(batch_num_tax), 1).at[0].set(0)
    root_idx = (root + tmp[:, None]).reshape(-1)
    local = _gather_rows(tax_h, root_idx, chunk=320).reshape(B, FIRST, HID)
    valid = (root != -1)
    local = jnp.where(valid[:, :, None], local, 0.0)
    pn = _gather_rows(item_embed, jnp.concatenate([pos, neg]), chunk=64)
    pos_e, neg_e = pn[:B], pn[B:]
    return _final(local, g_int, pos_e, neg_e)
